# baseline (speedup 1.0000x reference)
import jax
import jax.numpy as jnp
from jax import lax
from jax.experimental import pallas as pl
from jax.experimental.pallas import tpu as pltpu

_N = 10000
_BLOCK = 2048
_STEPS = 5
_REM = _N - (_STEPS - 1) * _BLOCK

_DN_RT = (((1,), (1,)), ((), ()))


def _cell_body(x_ref, ht_ref, wzt_ref, wrt_ref, wht_ref, b_ref, wlt_ref,
               blt_ref, outt_ref, hnewt_ref):
    i = pl.program_id(0)
    d_in = x_ref.shape[1]
    wz = wzt_ref[0, 0] + wzt_ref[1, 0]
    wr = wrt_ref[0, 0] + wrt_ref[1, 0]
    wh = wht_ref[0, 0] + wht_ref[1, 0]
    w_all = jnp.concatenate([wz, wr, wh], axis=0)
    x_b = x_ref[...]

    def gru(xb, htb):
        h_nat = jnp.transpose(htb)
        gx = (lax.dot_general(xb, w_all[:, :d_in], _DN_RT,
                              preferred_element_type=jnp.float32)
              + b_ref[...])
        zr = jax.nn.sigmoid(
            gx[:, :64]
            + lax.dot_general(h_nat, w_all[:64, d_in:], _DN_RT,
                              preferred_element_type=jnp.float32))
        z = zr[:, :32]
        r = zr[:, 32:]
        htl = jnp.tanh(
            gx[:, 64:]
            + lax.dot_general(h_nat * r, wh[:, d_in:], _DN_RT,
                              preferred_element_type=jnp.float32))
        h_new = z * h_nat + (1.0 - z) * htl
        out = (lax.dot_general(jnp.maximum(h_new, 0.0), wlt_ref[...], _DN_RT,
                               preferred_element_type=jnp.float32)
               + blt_ref[...])
        return jnp.transpose(out), jnp.transpose(h_new)

    @pl.when(i < _STEPS - 1)
    def _full_chunk():
        cols = pl.ds(i * _BLOCK, _BLOCK)
        ot, hn = gru(x_b, ht_ref[:, cols])
        outt_ref[:, cols] = ot
        hnewt_ref[:, cols] = hn

    @pl.when(i == _STEPS - 1)
    def _rem_chunk():
        cols = pl.ds((_STEPS - 1) * _BLOCK, _REM)
        ot, hn = gru(x_b[:_REM], ht_ref[:, cols])
        outt_ref[:, cols] = ot
        hnewt_ref[:, cols] = hn


def kernel(x, edge_index, edge_weight, h, Wz, bz, Wr, br, Wh, bh, W_lin, b_lin):
    del edge_index, edge_weight
    d_hid = h.shape[1]
    d_out = W_lin.shape[1]
    ht = h.T
    wzt = jnp.transpose(Wz, (0, 1, 3, 2))
    wrt = jnp.transpose(Wr, (0, 1, 3, 2))
    wht = jnp.transpose(Wh, (0, 1, 3, 2))
    wlt = W_lin.T
    b_all = jnp.concatenate([bz, br, bh])[None]
    blt = b_lin[None]

    full = lambda a: pl.BlockSpec(a.shape, lambda i: (0,) * a.ndim)
    out_t, h_new_t = pl.pallas_call(
        _cell_body,
        grid=(_STEPS,),
        in_specs=[
            pl.BlockSpec((_BLOCK, x.shape[1]), lambda i: (i, 0)),
            full(ht), full(wzt), full(wrt), full(wht),
            full(b_all), full(wlt), full(blt),
        ],
        out_specs=[
            pl.BlockSpec((d_out, _N), lambda i: (0, 0)),
            pl.BlockSpec((d_hid, _N), lambda i: (0, 0)),
        ],
        out_shape=[
            jax.ShapeDtypeStruct((d_out, _N), jnp.float32),
            jax.ShapeDtypeStruct((d_hid, _N), jnp.float32),
        ],
        compiler_params=pltpu.CompilerParams(
            dimension_semantics=("arbitrary",),
        ),
    )(x, ht, wzt, wrt, wht, b_all, wlt, blt)
    return (out_t.T, h_new_t.T)
